# baseline (speedup 1.0000x reference)
import functools

import jax
import jax.numpy as jnp
from jax import lax
from jax.experimental import pallas as pl
from jax.experimental.pallas import tpu as pltpu
from jax.experimental.pallas import tpu_sc as plsc

N = 10000
E = 320000
D = 128
R = 8
NR = N * R
NC = 2
NS = 16
NW = NC * NS
LL = 16
C = 128
EP = 327680
EPW = EP // NW
NCHUNK_W = EPW // C
EPT = EP // NS
NCHUNK_T = EPT // C
NRP = 81920
BINS_T = NRP // NS
BINS_W = NRP // NW
NQ = 3
WROWS = 3456
DT = WROWS // NS
ND = NQ * WROWS
BN = 1000
BNR = 1024

_mesh = plsc.VectorSubcoreMesh(
    core_axis_name="c", subcore_axis_name="s", num_cores=NC, num_subcores=NS)


@functools.partial(
    pl.kernel,
    out_type=jax.ShapeDtypeStruct((NRP,), jnp.float32),
    mesh=_mesh,
    scratch_types=[
        pltpu.VMEM_SHARED((NRP,), jnp.float32),
        pltpu.VMEM((BINS_W,), jnp.float32),
        pltpu.VMEM((C,), jnp.float32),
        pltpu.VMEM((C,), jnp.int32),
    ],
)
def _count_kernel(cidx_hbm, cnt_hbm, cnt_sh, buf_v, ones_v, cidx_v):
    c = lax.axis_index("c")
    s = lax.axis_index("s")
    wid = c * NS + s
    zero16 = jnp.zeros((LL,), jnp.float32)
    ones16 = jnp.ones((LL,), jnp.float32)

    def zf_body(i, carry):
        buf_v[pl.ds(i * LL, LL)] = zero16
        return carry
    lax.fori_loop(0, BINS_W // LL, zf_body, 0)
    for i in range(BINS_T // BINS_W):
        pltpu.sync_copy(buf_v, cnt_sh.at[pl.ds(s * BINS_T + i * BINS_W,
                                               BINS_W)])
    plsc.subcore_barrier()

    def of_body(i, carry):
        ones_v[pl.ds(i * LL, LL)] = ones16
        return carry
    lax.fori_loop(0, C // LL, of_body, 0)

    def h_body(i, carry):
        base = s * EPT + i * C
        pltpu.sync_copy(cidx_hbm.at[pl.ds(base, C)], cidx_v)
        pltpu.sync_copy(ones_v, cnt_sh.at[cidx_v], add=True)
        return carry
    lax.fori_loop(0, NCHUNK_T, h_body, 0)
    plsc.subcore_barrier()

    pltpu.sync_copy(cnt_sh.at[pl.ds(wid * BINS_W, BINS_W)], buf_v)
    pltpu.sync_copy(buf_v, cnt_hbm.at[pl.ds(wid * BINS_W, BINS_W)])


def _recip_body(c_ref, t_ref):
    row = (lax.broadcasted_iota(jnp.int32, (BNR, 1), 0)
           + pl.program_id(0) * BNR)
    w = jnp.where(row < NR, 1.0 / jnp.maximum(c_ref[...], 1.0), 0.0)
    t_ref[...] = jnp.broadcast_to(w, (BNR, D))


_recip = pl.pallas_call(
    _recip_body,
    grid=(NRP // BNR,),
    in_specs=[pl.BlockSpec((BNR, 1), lambda i: (i, 0))],
    out_specs=pl.BlockSpec((BNR, D), lambda i: (i, 0)),
    out_shape=jax.ShapeDtypeStruct((NRP, D), jnp.float32),
)


@functools.partial(
    pl.kernel,
    out_type=jax.ShapeDtypeStruct((NC * ND, D), jnp.float32),
    mesh=_mesh,
    scratch_types=[
        pltpu.VMEM_SHARED((WROWS, D), jnp.float32),
        pltpu.VMEM((C, D), jnp.float32),
        pltpu.VMEM((C, D), jnp.float32),
        pltpu.VMEM((C, D), jnp.float32),
        pltpu.VMEM((C, D), jnp.float32),
        pltpu.VMEM((C,), jnp.int32),
        pltpu.VMEM((C,), jnp.int32),
        pltpu.VMEM((C,), jnp.int32),
        pltpu.VMEM((C,), jnp.int32),
        pltpu.VMEM((C,), jnp.int32),
        pltpu.VMEM((C,), jnp.int32),
        pltpu.VMEM((C,), jnp.int32),
        pltpu.SemaphoreType.DMA,
        pltpu.SemaphoreType.DMA,
    ],
)
def _scatter_kernel(z_hbm, gidx_hbm, dst_hbm, cidx_hbm, table_hbm, part_hbm,
                    acc_sh, rows0_v, rows1_v, w0_v, w1_v, gidx0_v, gidx1_v,
                    dst_v, cidx0_v, cidx1_v, loc0_v, loc1_v, sem0, sem1):
    c = lax.axis_index("c")
    s = lax.axis_index("s")
    wid = c * NS + s
    zero16 = jnp.zeros((LL,), jnp.float32)
    rows_b = (rows0_v, rows1_v)
    w_b = (w0_v, w1_v)
    gidx_b = (gidx0_v, gidx1_v)
    cidx_b = (cidx0_v, cidx1_v)
    loc_b = (loc0_v, loc1_v)
    sem_b = (sem0, sem1)

    def _load_idx(q, chunk, b):
        base = wid * EPW + chunk * C
        pltpu.sync_copy(gidx_hbm.at[pl.ds(base, C)], gidx_b[b])
        pltpu.sync_copy(dst_hbm.at[pl.ds(base, C)], dst_v)
        pltpu.sync_copy(cidx_hbm.at[pl.ds(base, C)], cidx_b[b])
        for g in range(C // LL):
            sl = pl.ds(g * LL, LL)
            lv = dst_v[sl] - (q * WROWS)
            ok = (lv >= 0) & (lv < WROWS)
            loc_b[b][sl] = jnp.where(ok, lv, 0)
            cidx_b[b][sl] = jnp.where(ok, cidx_b[b][sl], NR)

    def _start(b):
        pltpu.async_copy(table_hbm.at[cidx_b[b]], w_b[b], sem_b[b])
        pltpu.async_copy(z_hbm.at[gidx_b[b]], rows_b[b], sem_b[b])

    def _finish(b):
        pltpu.make_async_copy(table_hbm.at[pl.ds(0, C)], w_b[b],
                              sem_b[b]).wait()
        pltpu.make_async_copy(z_hbm.at[pl.ds(0, C)], rows_b[b],
                              sem_b[b]).wait()

    for q in range(NQ):
        def zr_body(i, carry):
            for k in range(D // LL):
                rows0_v[i, pl.ds(k * LL, LL)] = zero16
            return carry
        lax.fori_loop(0, C, zr_body, 0)
        pltpu.sync_copy(rows0_v, acc_sh.at[pl.ds(s * DT, C)])
        pltpu.sync_copy(rows0_v.at[pl.ds(0, DT - C)],
                        acc_sh.at[pl.ds(s * DT + C, DT - C)])
        plsc.subcore_barrier()

        _load_idx(q, 0, 0)
        _start(0)

        def e_body(i, carry):
            for b in range(2):
                chunk = 2 * i + b
                ob = 1 - b

                @pl.when(chunk + 1 < NCHUNK_W)
                def _():
                    _load_idx(q, chunk + 1, ob)
                    _start(ob)

                _finish(b)
                rv = rows_b[b]
                wv = w_b[b]

                def s_body(j, carry2):
                    for k in range(D // LL):
                        sl = pl.ds(k * LL, LL)
                        rv[j, sl] = rv[j, sl] * wv[j, sl]
                    return carry2
                lax.fori_loop(0, C, s_body, 0)
                pltpu.sync_copy(rv, acc_sh.at[loc_b[b]], add=True)
            return carry
        lax.fori_loop(0, NCHUNK_W // 2, e_body, 0)
        plsc.subcore_barrier()

        r0 = s * DT
        pltpu.sync_copy(acc_sh.at[pl.ds(r0, C)],
                        part_hbm.at[pl.ds(c * ND + q * WROWS + r0, C)])
        pltpu.sync_copy(acc_sh.at[pl.ds(r0 + C, DT - C)],
                        part_hbm.at[pl.ds(c * ND + q * WROWS + r0 + C,
                                          DT - C)])
        plsc.subcore_barrier()


def _mm1_body(x_ref, w_ref, root_ref, b_ref, z_ref, out0_ref):
    x = x_ref[...]
    for r in range(R):
        z_ref[r] = jnp.dot(x, w_ref[r], preferred_element_type=jnp.float32)
    out0_ref[...] = (
        jnp.dot(x, root_ref[...], preferred_element_type=jnp.float32)
        + b_ref[...])


_mm1 = pl.pallas_call(
    _mm1_body,
    grid=(N // BN,),
    in_specs=[
        pl.BlockSpec((BN, D), lambda i: (i, 0)),
        pl.BlockSpec((R, D, D), lambda i: (0, 0, 0)),
        pl.BlockSpec((D, D), lambda i: (0, 0)),
        pl.BlockSpec((1, D), lambda i: (0, 0)),
    ],
    out_specs=(
        pl.BlockSpec((R, BN, D), lambda i: (0, i, 0)),
        pl.BlockSpec((BN, D), lambda i: (i, 0)),
    ),
    out_shape=(
        jax.ShapeDtypeStruct((R, N, D), jnp.float32),
        jax.ShapeDtypeStruct((N, D), jnp.float32),
    ),
)


def _mm2_body(y_ref, p_ref, w_ref, root_ref, b_ref, z_ref, out0_ref):
    h = jnp.maximum(y_ref[...] + p_ref[0] + p_ref[1], 0.0)
    for r in range(R):
        z_ref[r] = jnp.dot(h, w_ref[r], preferred_element_type=jnp.float32)
    out0_ref[...] = (
        jnp.dot(h, root_ref[...], preferred_element_type=jnp.float32)
        + b_ref[...])


_mm2 = pl.pallas_call(
    _mm2_body,
    grid=(N // BN,),
    in_specs=[
        pl.BlockSpec((BN, D), lambda i: (i, 0)),
        pl.BlockSpec((2, BN, D), lambda i: (0, i, 0)),
        pl.BlockSpec((R, D, D), lambda i: (0, 0, 0)),
        pl.BlockSpec((D, D), lambda i: (0, 0)),
        pl.BlockSpec((1, D), lambda i: (0, 0)),
    ],
    out_specs=(
        pl.BlockSpec((R, BN, D), lambda i: (0, i, 0)),
        pl.BlockSpec((BN, D), lambda i: (i, 0)),
    ),
    out_shape=(
        jax.ShapeDtypeStruct((R, N, D), jnp.float32),
        jax.ShapeDtypeStruct((N, D), jnp.float32),
    ),
)


def _add_body(y_ref, p_ref, o_ref):
    o_ref[...] = y_ref[...] + p_ref[0] + p_ref[1]


_add = pl.pallas_call(
    _add_body,
    grid=(N // BN,),
    in_specs=[
        pl.BlockSpec((BN, D), lambda i: (i, 0)),
        pl.BlockSpec((2, BN, D), lambda i: (0, i, 0)),
    ],
    out_specs=pl.BlockSpec((BN, D), lambda i: (i, 0)),
    out_shape=jax.ShapeDtypeStruct((N, D), jnp.float32),
)


def kernel(X, A, edge_type, W1, root1, b1, W2, root2, b2):
    src = A[0]
    dst = A[1]
    et = edge_type
    pad = EP - E
    gidx = jnp.concatenate([et * N + src, jnp.zeros((pad,), jnp.int32)])
    cidx = jnp.concatenate([dst * R + et, jnp.full((pad,), NR, jnp.int32)])
    dst_p = jnp.concatenate([dst, jnp.zeros((pad,), jnp.int32)])

    cnt = _count_kernel(cidx)
    table = _recip(cnt.reshape(NRP, 1))

    z1, out01 = _mm1(X, W1, root1, b1.reshape(1, D))
    p1 = _scatter_kernel(z1.reshape(R * N, D), gidx, dst_p, cidx, table)
    z2, out02 = _mm2(out01, p1.reshape(NC, ND, D)[:, :N], W2, root2,
                     b2.reshape(1, D))
    p2 = _scatter_kernel(z2.reshape(R * N, D), gidx, dst_p, cidx, table)
    return _add(out02, p2.reshape(NC, ND, D)[:, :N])


# speedup vs baseline: 14.3238x; 14.3238x over previous
import functools

import jax
import jax.numpy as jnp
from jax import lax
from jax.experimental import pallas as pl
from jax.experimental.pallas import tpu as pltpu
from jax.experimental.pallas import tpu_sc as plsc

N = 10000
E = 320000
D = 128
R = 8
NR = N * R
NC = 2
NS = 16
NW = NC * NS
LL = 16
C = 128
EP = 323584
EPW = EP // NW
NCHUNK_W = EPW // C
EPT = EP // NS
NCHUNK_T = EPT // C
NRP = 81920
BINS_T = NRP // NS
BINS_W = NRP // NW
NQ = 3
WROWS = 3456
TRASH = 3520
AROWS = 3584
ZT = AROWS // NS
DT = WROWS // NS
ND = NQ * WROWS
BN = 1000
BNR = 1024

_mesh = plsc.VectorSubcoreMesh(
    core_axis_name="c", subcore_axis_name="s", num_cores=NC, num_subcores=NS)


@functools.partial(
    pl.kernel,
    out_type=jax.ShapeDtypeStruct((NRP,), jnp.float32),
    mesh=_mesh,
    scratch_types=[
        pltpu.VMEM_SHARED((NRP,), jnp.float32),
        pltpu.VMEM((BINS_W,), jnp.float32),
        pltpu.VMEM((C,), jnp.float32),
        pltpu.VMEM((C,), jnp.int32),
    ],
)
def _count_kernel(cidx_hbm, cnt_hbm, cnt_sh, buf_v, ones_v, cidx_v):
    c = lax.axis_index("c")
    s = lax.axis_index("s")
    wid = c * NS + s
    zero16 = jnp.zeros((LL,), jnp.float32)
    ones16 = jnp.ones((LL,), jnp.float32)

    def zf_body(i, carry):
        buf_v[pl.ds(i * LL, LL)] = zero16
        return carry
    lax.fori_loop(0, BINS_W // LL, zf_body, 0)
    for i in range(BINS_T // BINS_W):
        pltpu.sync_copy(buf_v, cnt_sh.at[pl.ds(s * BINS_T + i * BINS_W,
                                               BINS_W)])
    plsc.subcore_barrier()

    def of_body(i, carry):
        ones_v[pl.ds(i * LL, LL)] = ones16
        return carry
    lax.fori_loop(0, C // LL, of_body, 0)

    def h_body(i, carry):
        base = s * EPT + i * C
        pltpu.sync_copy(cidx_hbm.at[pl.ds(base, C)], cidx_v)
        pltpu.sync_copy(ones_v, cnt_sh.at[cidx_v], add=True)
        return carry
    lax.fori_loop(0, NCHUNK_T, h_body, 0)
    plsc.subcore_barrier()

    pltpu.sync_copy(cnt_sh.at[pl.ds(wid * BINS_W, BINS_W)], buf_v)
    pltpu.sync_copy(buf_v, cnt_hbm.at[pl.ds(wid * BINS_W, BINS_W)])


def _recip_body(c_ref, t_ref):
    t_ref[...] = jnp.broadcast_to(1.0 / jnp.maximum(c_ref[...], 1.0),
                                  (BNR, D))


_recip = pl.pallas_call(
    _recip_body,
    grid=(NRP // BNR,),
    in_specs=[pl.BlockSpec((BNR, 1), lambda i: (i, 0))],
    out_specs=pl.BlockSpec((BNR, D), lambda i: (i, 0)),
    out_shape=jax.ShapeDtypeStruct((NRP, D), jnp.float32),
)


@functools.partial(
    pl.kernel,
    out_type=jax.ShapeDtypeStruct((NC * ND, D), jnp.float32),
    mesh=_mesh,
    scratch_types=[
        pltpu.VMEM_SHARED((AROWS, D), jnp.float32),
        pltpu.VMEM((C, D), jnp.float32),
        pltpu.VMEM((C, D), jnp.float32),
        pltpu.VMEM((3 * C,), jnp.int32),
        pltpu.VMEM((C,), jnp.int32),
        pltpu.SemaphoreType.DMA,
    ],
)
def _scatter_kernel(z_hbm, idx3_hbm, table_hbm, part_hbm,
                    acc_sh, rows_v, w_v, idx3_v, loc_v, sem):
    c = lax.axis_index("c")
    s = lax.axis_index("s")
    wid = c * NS + s
    zero16 = jnp.zeros((LL,), jnp.float32)

    for q in range(NQ):
        def zr_body(i, carry):
            for k in range(D // LL):
                rows_v[i, pl.ds(k * LL, LL)] = zero16
            return carry
        lax.fori_loop(0, C, zr_body, 0)
        pltpu.sync_copy(rows_v, acc_sh.at[pl.ds(s * ZT, C)])
        pltpu.sync_copy(rows_v.at[pl.ds(0, ZT - C)],
                        acc_sh.at[pl.ds(s * ZT + C, ZT - C)])
        plsc.subcore_barrier()

        def e_body(i, carry):
            base3 = (wid * NCHUNK_W + i) * (3 * C)
            pltpu.sync_copy(idx3_hbm.at[pl.ds(base3, 3 * C)], idx3_v)
            cp_w = pltpu.async_copy(table_hbm.at[idx3_v.at[pl.ds(2 * C, C)]],
                                    w_v, sem)
            cp_z = pltpu.async_copy(z_hbm.at[idx3_v.at[pl.ds(0, C)]],
                                    rows_v, sem)

            for g in range(C // LL):
                sl = pl.ds(g * LL, LL)
                dv = idx3_v[pl.ds(C + g * LL, LL)]
                lv = dv - (q * WROWS)
                ok = (lv >= 0) & (lv < WROWS)
                loc_v[sl] = jnp.where(ok, lv, TRASH)

            cp_w.wait()
            cp_z.wait()

            def s_body(j, carry2):
                for k in range(D // LL):
                    sl = pl.ds(k * LL, LL)
                    rows_v[j, sl] = rows_v[j, sl] * w_v[j, sl]
                return carry2
            lax.fori_loop(0, C, s_body, 0)
            pltpu.sync_copy(rows_v, acc_sh.at[loc_v], add=True)
            return carry
        lax.fori_loop(0, NCHUNK_W, e_body, 0)
        plsc.subcore_barrier()

        r0 = s * DT
        pltpu.sync_copy(acc_sh.at[pl.ds(r0, C)],
                        part_hbm.at[pl.ds(c * ND + q * WROWS + r0, C)])
        pltpu.sync_copy(acc_sh.at[pl.ds(r0 + C, DT - C)],
                        part_hbm.at[pl.ds(c * ND + q * WROWS + r0 + C,
                                          DT - C)])
        plsc.subcore_barrier()


def _mm1_body(x_ref, w_ref, root_ref, b_ref, z_ref, out0_ref):
    x = x_ref[...]
    for r in range(R):
        z_ref[r] = jnp.dot(x, w_ref[r], preferred_element_type=jnp.float32)
    out0_ref[...] = (
        jnp.dot(x, root_ref[...], preferred_element_type=jnp.float32)
        + b_ref[...])


_mm1 = pl.pallas_call(
    _mm1_body,
    grid=(N // BN,),
    in_specs=[
        pl.BlockSpec((BN, D), lambda i: (i, 0)),
        pl.BlockSpec((R, D, D), lambda i: (0, 0, 0)),
        pl.BlockSpec((D, D), lambda i: (0, 0)),
        pl.BlockSpec((1, D), lambda i: (0, 0)),
    ],
    out_specs=(
        pl.BlockSpec((R, BN, D), lambda i: (0, i, 0)),
        pl.BlockSpec((BN, D), lambda i: (i, 0)),
    ),
    out_shape=(
        jax.ShapeDtypeStruct((R, N, D), jnp.float32),
        jax.ShapeDtypeStruct((N, D), jnp.float32),
    ),
)


def _mm2_body(y_ref, p_ref, w_ref, root_ref, b_ref, z_ref, out0_ref):
    h = jnp.maximum(y_ref[...] + p_ref[0] + p_ref[1], 0.0)
    for r in range(R):
        z_ref[r] = jnp.dot(h, w_ref[r], preferred_element_type=jnp.float32)
    out0_ref[...] = (
        jnp.dot(h, root_ref[...], preferred_element_type=jnp.float32)
        + b_ref[...])


_mm2 = pl.pallas_call(
    _mm2_body,
    grid=(N // BN,),
    in_specs=[
        pl.BlockSpec((BN, D), lambda i: (i, 0)),
        pl.BlockSpec((2, BN, D), lambda i: (0, i, 0)),
        pl.BlockSpec((R, D, D), lambda i: (0, 0, 0)),
        pl.BlockSpec((D, D), lambda i: (0, 0)),
        pl.BlockSpec((1, D), lambda i: (0, 0)),
    ],
    out_specs=(
        pl.BlockSpec((R, BN, D), lambda i: (0, i, 0)),
        pl.BlockSpec((BN, D), lambda i: (i, 0)),
    ),
    out_shape=(
        jax.ShapeDtypeStruct((R, N, D), jnp.float32),
        jax.ShapeDtypeStruct((N, D), jnp.float32),
    ),
)


def _add_body(y_ref, p_ref, o_ref):
    o_ref[...] = y_ref[...] + p_ref[0] + p_ref[1]


_add = pl.pallas_call(
    _add_body,
    grid=(N // BN,),
    in_specs=[
        pl.BlockSpec((BN, D), lambda i: (i, 0)),
        pl.BlockSpec((2, BN, D), lambda i: (0, i, 0)),
    ],
    out_specs=pl.BlockSpec((BN, D), lambda i: (i, 0)),
    out_shape=jax.ShapeDtypeStruct((N, D), jnp.float32),
)


def kernel(X, A, edge_type, W1, root1, b1, W2, root2, b2):
    src = A[0]
    dst = A[1]
    et = edge_type
    pad = EP - E
    gidx = jnp.concatenate([et * N + src, jnp.zeros((pad,), jnp.int32)])
    cidx = jnp.concatenate([dst * R + et, jnp.full((pad,), NR, jnp.int32)])
    dst_p = jnp.concatenate([dst, jnp.full((pad,), N, jnp.int32)])
    idx3 = jnp.stack([gidx.reshape(-1, C), dst_p.reshape(-1, C),
                      cidx.reshape(-1, C)], axis=1).reshape(-1)

    cnt = _count_kernel(cidx)
    table = _recip(cnt.reshape(NRP, 1))

    z1, out01 = _mm1(X, W1, root1, b1.reshape(1, D))
    p1 = _scatter_kernel(z1.reshape(R * N, D), idx3, table)
    z2, out02 = _mm2(out01, p1.reshape(NC, ND, D)[:, :N], W2, root2,
                     b2.reshape(1, D))
    p2 = _scatter_kernel(z2.reshape(R * N, D), idx3, table)
    return _add(out02, p2.reshape(NC, ND, D)[:, :N])
